# baseline (speedup 1.0000x reference)
import functools

import numpy as np
import jax
import jax.numpy as jnp
from jax.experimental import pallas as pl

_VOCAB = 1000
_SCHEDULE = {}


def _uniform_body(col, out_ref):
    a, s, l = out_ref.shape
    r = jax.lax.broadcasted_iota(jnp.int32, (1, s, l), 1)
    c = jax.lax.broadcasted_iota(jnp.int32, (1, s, l), 2)
    vocab_col = jax.lax.rem(r * l + c, _VOCAB)
    slab = jnp.where(vocab_col == col, 10.0, -10.0)
    out_ref[...] = jnp.broadcast_to(slab, (a, s, l))


def _general_body(col_ref, out_ref):
    _, bt, v = out_ref.shape
    lane = jax.lax.broadcasted_iota(jnp.int32, (1, bt, v), 2)
    out_ref[...] = jnp.where(lane == col_ref[...][None], 10.0, -10.0)


def kernel(input_ids, anchor):
    B, T = input_ids.shape
    past_len = 0
    cols_np = np.array(
        [int(_SCHEDULE.get(past_len + t, 1)) for t in range(T)], dtype=np.int32
    )
    out_shape = jax.ShapeDtypeStruct((B, T, _VOCAB), jnp.float32)

    total = B * T * _VOCAB
    lanes = 128
    slab_rows = 1000
    a_total = total // (slab_rows * lanes)
    ba = 8
    if bool((cols_np == cols_np[0]).all()) and a_total % ba == 0:
        out = pl.pallas_call(
            functools.partial(_uniform_body, int(cols_np[0])),
            grid=(a_total // ba,),
            out_specs=pl.BlockSpec((ba, slab_rows, lanes), lambda i: (i, 0, 0)),
            out_shape=jax.ShapeDtypeStruct((a_total, slab_rows, lanes), jnp.float32),
        )()
        return out.reshape(B, T, _VOCAB)
    bt = 1024
    cols = jnp.asarray(cols_np.reshape(T, 1))
    return pl.pallas_call(
        _general_body,
        grid=(B, T // bt),
        in_specs=[pl.BlockSpec((bt, 1), lambda b, j: (j, 0))],
        out_specs=pl.BlockSpec((1, bt, _VOCAB), lambda b, j: (b, j, 0)),
        out_shape=out_shape,
    )(cols)


# speedup vs baseline: 1.5841x; 1.5841x over previous
import functools

import numpy as np
import jax
import jax.numpy as jnp
from jax.experimental import pallas as pl

_VOCAB = 1000
_SCHEDULE = {}
_LANES = 128


def _uniform_body(col, out_ref):
    bt, w = out_ref.shape
    j = pl.program_id(1)
    lane = jax.lax.broadcasted_iota(jnp.int32, (8, w), 1) + j * _LANES
    rows8 = jnp.where(lane == col, 10.0, -10.0)
    out_ref[...] = jnp.broadcast_to(rows8[:1], (bt, w))


def _general_body(col_ref, out_ref):
    bt, w = out_ref.shape
    j = pl.program_id(2)
    lane = jax.lax.broadcasted_iota(jnp.int32, (bt, w), 1) + j * _LANES
    out_ref[...] = jnp.where(lane == col_ref[...], 10.0, -10.0)


def kernel(input_ids, anchor):
    B, T = input_ids.shape
    past_len = 0
    cols_np = np.array(
        [int(_SCHEDULE.get(past_len + t, 1)) for t in range(T)], dtype=np.int32
    )
    rows = B * T
    ncol = (_VOCAB + _LANES - 1) // _LANES
    if bool((cols_np == cols_np[0]).all()):
        bt = 2048
        out = pl.pallas_call(
            functools.partial(_uniform_body, int(cols_np[0])),
            grid=(rows // bt, ncol),
            out_specs=pl.BlockSpec((bt, _LANES), lambda i, j: (i, j)),
            out_shape=jax.ShapeDtypeStruct((rows, _VOCAB), jnp.float32),
        )()
        return out.reshape(B, T, _VOCAB)
    bt = 1024
    cols = jnp.asarray(np.tile(cols_np, B).reshape(rows, 1))
    out = pl.pallas_call(
        _general_body,
        grid=(1, rows // bt, ncol),
        in_specs=[pl.BlockSpec((bt, 1), lambda _, i, j: (i, 0))],
        out_specs=pl.BlockSpec((bt, _LANES), lambda _, i, j: (i, j)),
        out_shape=jax.ShapeDtypeStruct((rows, _VOCAB), jnp.float32),
    )(cols)
    return out.reshape(B, T, _VOCAB)


# speedup vs baseline: 1.6957x; 1.0704x over previous
import functools

import numpy as np
import jax
import jax.numpy as jnp
from jax import lax
from jax.experimental import pallas as pl
from jax.experimental.pallas import tpu as pltpu
from jax.experimental.pallas import tpu_sc as plsc

_VOCAB = 1000
_SCHEDULE = {}
_NC = 2
_NS = 16
_CR = 16


def _make_sc_fill(rows, col):
    per_w = rows // (_NC * _NS)
    mesh = plsc.VectorSubcoreMesh(core_axis_name="c", subcore_axis_name="s")

    @functools.partial(
        pl.kernel,
        mesh=mesh,
        out_type=jax.ShapeDtypeStruct((rows, _VOCAB), jnp.float32),
        scratch_types=[
            pltpu.VMEM((_CR, _VOCAB), jnp.float32),
            pltpu.SemaphoreType.DMA,
        ],
    )
    def sc_fill(out_hbm, buf, sem):
        wid = lax.axis_index("s") * _NC + lax.axis_index("c")
        starts = [16 * j for j in range(_VOCAB // 16)] + [_VOCAB - 16]
        for c0 in starts:
            colv = lax.iota(jnp.int32, 16) + c0
            seg = jnp.where(colv == col, 10.0, -10.0)
            for r in range(_CR):
                buf[r, pl.ds(c0, 16)] = seg
        base = wid * per_w
        copies = [
            pltpu.make_async_copy(
                buf, out_hbm.at[pl.ds(base + k * _CR, _CR), :], sem
            )
            for k in range(per_w // _CR)
        ]
        for cp in copies:
            cp.start()
        for cp in copies:
            cp.wait()

    return sc_fill


def _general_body(col_ref, out_ref):
    bt, v = out_ref.shape
    lane = jax.lax.broadcasted_iota(jnp.int32, (bt, v), 1)
    out_ref[...] = jnp.where(lane == col_ref[...], 10.0, -10.0)


def kernel(input_ids, anchor):
    B, T = input_ids.shape
    past_len = 0
    cols_np = np.array(
        [int(_SCHEDULE.get(past_len + t, 1)) for t in range(T)], dtype=np.int32
    )
    rows = B * T
    if bool((cols_np == cols_np[0]).all()):
        out = _make_sc_fill(rows, int(cols_np[0]))()
        return out.reshape(B, T, _VOCAB)
    bt = 1024
    cols = jnp.asarray(np.tile(cols_np, B).reshape(rows, 1))
    out = pl.pallas_call(
        _general_body,
        grid=(rows // bt,),
        in_specs=[pl.BlockSpec((bt, 1), lambda i: (i, 0))],
        out_specs=pl.BlockSpec((bt, _VOCAB), lambda i: (i, 0)),
        out_shape=jax.ShapeDtypeStruct((rows, _VOCAB), jnp.float32),
    )(cols)
    return out.reshape(B, T, _VOCAB)


# speedup vs baseline: 1.6979x; 1.0013x over previous
import functools

import numpy as np
import jax
import jax.numpy as jnp
from jax import lax
from jax.experimental import pallas as pl
from jax.experimental.pallas import tpu as pltpu
from jax.experimental.pallas import tpu_sc as plsc

_VOCAB = 1000
_SCHEDULE = {}
_NC = 2
_NS = 16
_CR = 16


def _make_sc_fill(rows, col):
    per_w = rows // (_NC * _NS)
    mesh = plsc.VectorSubcoreMesh(core_axis_name="c", subcore_axis_name="s")

    @functools.partial(
        pl.kernel,
        mesh=mesh,
        out_type=jax.ShapeDtypeStruct((rows, _VOCAB), jnp.float32),
        scratch_types=[
            pltpu.VMEM((_CR, _VOCAB), jnp.float32),
            pltpu.SemaphoreType.DMA,
        ],
        compiler_params=pltpu.CompilerParams(use_tc_tiling_on_sc=True),
    )
    def sc_fill(out_hbm, buf, sem):
        wid = lax.axis_index("s") * _NC + lax.axis_index("c")
        starts = [16 * j for j in range(_VOCAB // 16)] + [_VOCAB - 16]
        for c0 in starts:
            colv = lax.iota(jnp.int32, 16) + c0
            seg = jnp.where(colv == col, 10.0, -10.0)
            for r in range(_CR):
                buf[r, pl.ds(c0, 16)] = seg
        base = wid * per_w
        copies = [
            pltpu.make_async_copy(
                buf, out_hbm.at[pl.ds(base + k * _CR, _CR), :], sem
            )
            for k in range(per_w // _CR)
        ]
        for cp in copies:
            cp.start()
        for cp in copies:
            cp.wait()

    return sc_fill


def _general_body(col_ref, out_ref):
    bt, v = out_ref.shape
    lane = jax.lax.broadcasted_iota(jnp.int32, (bt, v), 1)
    out_ref[...] = jnp.where(lane == col_ref[...], 10.0, -10.0)


def kernel(input_ids, anchor):
    B, T = input_ids.shape
    past_len = 0
    cols_np = np.array(
        [int(_SCHEDULE.get(past_len + t, 1)) for t in range(T)], dtype=np.int32
    )
    rows = B * T
    if bool((cols_np == cols_np[0]).all()):
        out = _make_sc_fill(rows, int(cols_np[0]))()
        return out.reshape(B, T, _VOCAB)
    bt = 1024
    cols = jnp.asarray(np.tile(cols_np, B).reshape(rows, 1))
    out = pl.pallas_call(
        _general_body,
        grid=(rows // bt,),
        in_specs=[pl.BlockSpec((bt, 1), lambda i: (i, 0))],
        out_specs=pl.BlockSpec((bt, _VOCAB), lambda i: (i, 0)),
        out_shape=jax.ShapeDtypeStruct((rows, _VOCAB), jnp.float32),
    )(cols)
    return out.reshape(B, T, _VOCAB)


# speedup vs baseline: 1.7010x; 1.0018x over previous
import functools

import numpy as np
import jax
import jax.numpy as jnp
from jax import lax
from jax.experimental import pallas as pl
from jax.experimental.pallas import tpu as pltpu
from jax.experimental.pallas import tpu_sc as plsc

_VOCAB = 1000
_SCHEDULE = {}
_NC = 2
_NS = 16
_CR = 16


def _make_sc_fill(rows, col):
    per_w = rows // (_NC * _NS)
    mesh = plsc.VectorSubcoreMesh(core_axis_name="c", subcore_axis_name="s")

    @functools.partial(
        pl.kernel,
        mesh=mesh,
        out_type=jax.ShapeDtypeStruct((rows, _VOCAB), jnp.float32),
        scratch_types=[
            pltpu.VMEM((_CR, _VOCAB), jnp.float32),
            pltpu.SemaphoreType.DMA,
        ],
        compiler_params=pltpu.CompilerParams(
            use_tc_tiling_on_sc=True, needs_layout_passes=True
        ),
    )
    def sc_fill(out_hbm, buf, sem):
        wid = lax.axis_index("s") * _NC + lax.axis_index("c")
        starts = [16 * j for j in range(_VOCAB // 16)] + [_VOCAB - 16]
        for c0 in starts:
            colv = lax.iota(jnp.int32, 16) + c0
            seg = jnp.where(colv == col, 10.0, -10.0)
            for r in range(_CR):
                buf[r, pl.ds(c0, 16)] = seg
        base = wid * per_w
        copies = [
            pltpu.make_async_copy(
                buf, out_hbm.at[pl.ds(base + k * _CR, _CR), :], sem
            )
            for k in range(per_w // _CR)
        ]
        for cp in copies:
            cp.start()
        for cp in copies:
            cp.wait()

    return sc_fill


def _general_body(col_ref, out_ref):
    bt, v = out_ref.shape
    lane = jax.lax.broadcasted_iota(jnp.int32, (bt, v), 1)
    out_ref[...] = jnp.where(lane == col_ref[...], 10.0, -10.0)


def kernel(input_ids, anchor):
    B, T = input_ids.shape
    past_len = 0
    cols_np = np.array(
        [int(_SCHEDULE.get(past_len + t, 1)) for t in range(T)], dtype=np.int32
    )
    rows = B * T
    if bool((cols_np == cols_np[0]).all()):
        out = _make_sc_fill(rows, int(cols_np[0]))()
        return out.reshape(B, T, _VOCAB)
    bt = 1024
    cols = jnp.asarray(np.tile(cols_np, B).reshape(rows, 1))
    out = pl.pallas_call(
        _general_body,
        grid=(rows // bt,),
        in_specs=[pl.BlockSpec((bt, 1), lambda i: (i, 0))],
        out_specs=pl.BlockSpec((bt, _VOCAB), lambda i: (i, 0)),
        out_shape=jax.ShapeDtypeStruct((rows, _VOCAB), jnp.float32),
    )(cols)
    return out.reshape(B, T, _VOCAB)


# speedup vs baseline: 1.8308x; 1.0763x over previous
import functools

import numpy as np
import jax
import jax.numpy as jnp
from jax.experimental import pallas as pl
from jax.experimental.pallas import tpu as pltpu

_VOCAB = 1000
_SCHEDULE = {}
_BT = 512
_NSEM = 16


def _uniform_body(col, n_blocks, out_ref, scratch, *sems):
    bt, v = scratch.shape
    lane = jax.lax.broadcasted_iota(jnp.int32, (8, v), 1)
    rows8 = jnp.where(lane == col, 10.0, -10.0)
    scratch[...] = jnp.broadcast_to(rows8[:1], (bt, v))
    copies = [
        pltpu.make_async_copy(
            scratch, out_ref.at[pl.ds(i * bt, bt), :], sems[i % _NSEM]
        )
        for i in range(n_blocks)
    ]
    for c in copies:
        c.start()
    for c in copies:
        c.wait()


def _general_body(col_ref, out_ref):
    bt, v = out_ref.shape
    lane = jax.lax.broadcasted_iota(jnp.int32, (bt, v), 1)
    out_ref[...] = jnp.where(lane == col_ref[...], 10.0, -10.0)


def kernel(input_ids, anchor):
    B, T = input_ids.shape
    past_len = 0
    cols_np = np.array(
        [int(_SCHEDULE.get(past_len + t, 1)) for t in range(T)], dtype=np.int32
    )
    rows = B * T
    if bool((cols_np == cols_np[0]).all()):
        n_blocks = rows // _BT
        out = pl.pallas_call(
            functools.partial(_uniform_body, int(cols_np[0]), n_blocks),
            out_specs=pl.BlockSpec(memory_space=pl.ANY),
            out_shape=jax.ShapeDtypeStruct((rows, _VOCAB), jnp.float32),
            scratch_shapes=[pltpu.VMEM((_BT, _VOCAB), jnp.float32)]
            + [pltpu.SemaphoreType.DMA] * _NSEM,
        )()
        return out.reshape(B, T, _VOCAB)
    bt = 1024
    cols = jnp.asarray(np.tile(cols_np, B).reshape(rows, 1))
    out = pl.pallas_call(
        _general_body,
        grid=(rows // bt,),
        in_specs=[pl.BlockSpec((bt, 1), lambda i: (i, 0))],
        out_specs=pl.BlockSpec((bt, _VOCAB), lambda i: (i, 0)),
        out_shape=jax.ShapeDtypeStruct((rows, _VOCAB), jnp.float32),
    )(cols)
    return out.reshape(B, T, _VOCAB)
